# baseline (speedup 1.0000x reference)
import functools

import jax
import jax.numpy as jnp
from jax.experimental import pallas as pl
from jax.experimental.pallas import tpu as pltpu


def _fused_kernel(x_ref, w_ref, norm_ref, a_ref, norm_dst_ref, o_ref,
                  m_ref, *, num_divisions, num_heads, fout, m_chunk):
    n = x_ref.shape[0]
    hf = num_heads * fout

    for c in range(0, n, m_chunk):
        xn = x_ref[c:c + m_chunk, :] * norm_ref[c:c + m_chunk, :]
        m_ref[c:c + m_chunk, :] = jnp.dot(
            xn, w_ref[...], preferred_element_type=jnp.float32
        ).astype(m_ref.dtype)

    nrm = norm_dst_ref[...]
    for d in range(num_divisions):
        hd = jnp.dot(a_ref[d], m_ref[:, d * hf:(d + 1) * hf],
                     preferred_element_type=jnp.float32)
        hd = jnp.maximum(hd * nrm, 0.0)
        for h in range(num_heads):
            dst = (h * num_divisions + d) * fout
            o_ref[:, dst:dst + fout] = hd[:, h * fout:(h + 1) * fout]


def kernel(x, weights, adj, norm):
    N, Fin = x.shape
    H, D, _, Fout = weights.shape
    HF = H * Fout
    DHF = D * HF

    tile_dst = min(N, 256)

    w_ker = jnp.transpose(weights, (2, 1, 0, 3)).reshape(Fin, DHF)

    out = pl.pallas_call(
        functools.partial(_fused_kernel, num_divisions=D, num_heads=H,
                          fout=Fout, m_chunk=min(N, 512)),
        out_shape=jax.ShapeDtypeStruct((N, DHF), jnp.float32),
        grid=(N // tile_dst,),
        in_specs=[
            pl.BlockSpec((N, Fin), lambda i: (0, 0)),
            pl.BlockSpec((Fin, DHF), lambda i: (0, 0)),
            pl.BlockSpec((N, 1), lambda i: (0, 0)),
            pl.BlockSpec((D, tile_dst, N), lambda i: (0, i, 0)),
            pl.BlockSpec((tile_dst, 1), lambda i: (i, 0)),
        ],
        out_specs=pl.BlockSpec((tile_dst, DHF), lambda i: (i, 0)),
        scratch_shapes=[pltpu.VMEM((N, DHF), jnp.float32)],
        compiler_params=pltpu.CompilerParams(
            dimension_semantics=("parallel",)),
    )(x, w_ker, norm, adj, norm)

    return out


# speedup vs baseline: 1.0035x; 1.0035x over previous
import functools

import jax
import jax.numpy as jnp
from jax.experimental import pallas as pl
from jax.experimental.pallas import tpu as pltpu


def _fused_kernel(x_ref, w_ref, norm_ref, a_ref, norm_dst_ref, o_ref,
                  m_ref, *, num_divisions, num_heads, fout, m_chunk):
    n = x_ref.shape[0]
    hf = num_heads * fout

    for c in range(0, n, m_chunk):
        xn = x_ref[c:c + m_chunk, :] * norm_ref[c:c + m_chunk, :]
        m_ref[c:c + m_chunk, :] = jnp.dot(
            xn, w_ref[...], preferred_element_type=jnp.float32
        ).astype(m_ref.dtype)

    nrm = norm_dst_ref[...]
    for d in range(num_divisions):
        ad = a_ref[d].astype(jnp.bfloat16)
        hd = jnp.dot(ad, m_ref[:, d * hf:(d + 1) * hf],
                     preferred_element_type=jnp.float32)
        hd = jnp.maximum(hd * nrm, 0.0)
        for h in range(num_heads):
            dst = (h * num_divisions + d) * fout
            o_ref[:, dst:dst + fout] = hd[:, h * fout:(h + 1) * fout]


def kernel(x, weights, adj, norm):
    N, Fin = x.shape
    H, D, _, Fout = weights.shape
    HF = H * Fout
    DHF = D * HF

    tile_dst = min(N, 256)

    w_ker = jnp.transpose(weights, (2, 1, 0, 3)).reshape(Fin, DHF)

    out = pl.pallas_call(
        functools.partial(_fused_kernel, num_divisions=D, num_heads=H,
                          fout=Fout, m_chunk=min(N, 512)),
        out_shape=jax.ShapeDtypeStruct((N, DHF), jnp.float32),
        grid=(N // tile_dst,),
        in_specs=[
            pl.BlockSpec((N, Fin), lambda i: (0, 0)),
            pl.BlockSpec((Fin, DHF), lambda i: (0, 0)),
            pl.BlockSpec((N, 1), lambda i: (0, 0)),
            pl.BlockSpec((D, tile_dst, N), lambda i: (0, i, 0)),
            pl.BlockSpec((tile_dst, 1), lambda i: (i, 0)),
        ],
        out_specs=pl.BlockSpec((tile_dst, DHF), lambda i: (i, 0)),
        scratch_shapes=[pltpu.VMEM((N, DHF), jnp.bfloat16)],
        compiler_params=pltpu.CompilerParams(
            dimension_semantics=("parallel",)),
    )(x, w_ker, norm, adj, norm)

    return out
